# baseline (speedup 1.0000x reference)
import functools

import jax
import jax.numpy as jnp
from jax import lax
from jax.experimental import pallas as pl
from jax.experimental.pallas import tpu as pltpu
from jax.experimental.pallas import tpu_sc as plsc

BATCH = 16384
D = 64
NC, NS = 2, 16
NW = NC * NS
B_PER_W = BATCH // NW

_SC_MESH = plsc.VectorSubcoreMesh(core_axis_name="c", subcore_axis_name="s")


@functools.partial(
    pl.kernel,
    mesh=_SC_MESH,
    compiler_params=pltpu.CompilerParams(use_tc_tiling_on_sc=False),
    out_type=(
        jax.ShapeDtypeStruct((BATCH, D), jnp.float32),
        jax.ShapeDtypeStruct((BATCH, D), jnp.float32),
        jax.ShapeDtypeStruct((BATCH, D), jnp.float32),
        jax.ShapeDtypeStruct((BATCH, D), jnp.float32),
    ),
    scratch_types=[
        pltpu.VMEM((B_PER_W,), jnp.int32),
        pltpu.VMEM((B_PER_W,), jnp.int32),
        pltpu.VMEM((B_PER_W, D), jnp.float32),
        pltpu.VMEM((B_PER_W, D), jnp.float32),
        pltpu.SemaphoreType.DMA,
        pltpu.SemaphoreType.DMA,
        pltpu.SemaphoreType.DMA,
        pltpu.SemaphoreType.DMA,
    ],
)
def _gather4(uid_hbm, iid_hbm, umf_hbm, imf_hbm, uneu_hbm, ineu_hbm,
             out_umf, out_imf, out_uneu, out_ineu,
             idx_u, idx_i, buf_a, buf_b, sem_a, sem_b, sem_wa, sem_wb):
    wid = lax.axis_index("s") * NC + lax.axis_index("c")
    base = wid * B_PER_W
    pltpu.sync_copy(uid_hbm.at[pl.ds(base, B_PER_W)], idx_u)
    pltpu.sync_copy(iid_hbm.at[pl.ds(base, B_PER_W)], idx_i)
    cp_a = pltpu.async_copy(umf_hbm.at[idx_u], buf_a, sem_a)
    cp_b = pltpu.async_copy(imf_hbm.at[idx_i], buf_b, sem_b)
    cp_a.wait()
    wr_a = pltpu.async_copy(buf_a, out_umf.at[pl.ds(base, B_PER_W)], sem_wa)
    cp_b.wait()
    wr_b = pltpu.async_copy(buf_b, out_imf.at[pl.ds(base, B_PER_W)], sem_wb)
    wr_a.wait()
    cp_a = pltpu.async_copy(uneu_hbm.at[idx_u], buf_a, sem_a)
    wr_b.wait()
    cp_b = pltpu.async_copy(ineu_hbm.at[idx_i], buf_b, sem_b)
    cp_a.wait()
    wr_a = pltpu.async_copy(buf_a, out_uneu.at[pl.ds(base, B_PER_W)], sem_wa)
    cp_b.wait()
    wr_b = pltpu.async_copy(buf_b, out_ineu.at[pl.ds(base, B_PER_W)], sem_wb)
    wr_a.wait()
    wr_b.wait()


def _mlp_body(umf_ref, imf_ref, uneu_ref, ineu_ref,
              w0a_ref, w0b_ref, b0_ref, w1t_ref, b1_ref, out_ref):
    h0 = jax.nn.sigmoid(
        jnp.dot(uneu_ref[...], w0a_ref[...], preferred_element_type=jnp.float32)
        + jnp.dot(ineu_ref[...], w0b_ref[...], preferred_element_type=jnp.float32)
        + b0_ref[...])
    h1 = jax.nn.sigmoid(
        jnp.dot(h0, w1t_ref[...], preferred_element_type=jnp.float32)
        + b1_ref[...])
    gmf = jnp.sum(umf_ref[...] * imf_ref[...], axis=1)
    out_ref[...] = gmf + jnp.sum(h1, axis=1)


_BLK = 2048


def _mlp(umf, imf, uneu, ineu, w0a, w0b, b0, w1t, b1):
    grid = (BATCH // _BLK,)
    row_spec = pl.BlockSpec((_BLK, D), lambda i: (i, 0))
    full = lambda shape: pl.BlockSpec(shape, lambda i: (0,) * len(shape))
    return pl.pallas_call(
        _mlp_body,
        grid=grid,
        in_specs=[
            row_spec, row_spec, row_spec, row_spec,
            full((D, 128)), full((D, 128)), full((1, 128)),
            full((128, 64)), full((1, 64)),
        ],
        out_specs=pl.BlockSpec((_BLK,), lambda i: (i,)),
        out_shape=jax.ShapeDtypeStruct((BATCH,), jnp.float32),
    )(umf, imf, uneu, ineu, w0a, w0b, b0, w1t, b1)


def kernel(user_id, item_id, users_mf, items_mf, users_neu, items_neu,
           W0, b0, W1, b1):
    uid = user_id.astype(jnp.int32)
    iid = item_id.astype(jnp.int32)
    umf, imf, uneu, ineu = _gather4(uid, iid, users_mf, items_mf,
                                    users_neu, items_neu)
    w0a = W0[:, :D].T
    w0b = W0[:, D:].T
    w1t = W1.T
    return _mlp(umf, imf, uneu, ineu, w0a, w0b,
                b0.reshape(1, -1), w1t, b1.reshape(1, -1))
